# Initial kernel scaffold; baseline (speedup 1.0000x reference)
import jax
import jax.numpy as jnp
from jax.experimental import pallas as pl


def kernel(assmpt_feat, rule_feat, non_assmpt_feat, W_emb, b_emb, Wc, bc, ln_g, ln_b, Wcls, bcls, edges_src, edges_dst):
    raise NotImplementedError("write your pallas kernel here")


# speedup vs baseline: 5.5088x; 5.5088x over previous
import functools

import jax
import jax.numpy as jnp
from jax import lax
from jax.experimental import pallas as pl
from jax.experimental.pallas import tpu as pltpu
from jax.experimental.pallas import tpu_sc as plsc

_N_TYPES = (20000, 15000, 15000)
_ETYPES = ((0, 1), (2, 1), (2, 0), (0, 0), (1, 2), (1, 0), (0, 0), (1, 1), (2, 2))
_E = 90000
_D_IN, _D_HID, _D_OUT, _N_LAYERS = 128, 64, 8, 3

_NC, _NS = 2, 16
_CH = 128
_EPAD = 90112
_NROW = _EPAD // _CH
_RPT = _NROW // _NS

_SRC_GROUPS = ((0, 3, 6), (4, 5, 7), (1, 2, 8))
_SLAB = {e: k for grp in _SRC_GROUPS for k, e in enumerate(grp)}
_DST_GROUPS = ((2, 3, 5, 6), (0, 1, 7), (4, 8))

_JOBS = (
    ((0, 0, 44), (2, 0, 44), (4, 0, 44), (6, 0, 44), (8, 0, 24)),
    ((1, 0, 44), (3, 0, 44), (5, 0, 44), (7, 0, 44), (8, 24, 20)),
)
_JOB_OUT = ((0, 1, 2, 3, 4), (5, 6, 7, 8, 9))
_DST_CONTRIBS = (
    ((1, 2), (3, 6), (6, 3), (7, 5)),
    ((0, 0), (5, 1), (8, 7)),
    ((2, 4), (4, 8), (9, 8)),
)

_ZCH = 176


def _agg_rows(n):
    q = -(-(n + 1) // _NS)
    q = -(-q // _ZCH) * _ZCH
    return _NS * q


def _deg_rows(n):
    q = -(-(n + 1) // _NS)
    q = -(-q // 128) * 128
    return _NS * q


def _feat_scatter_body(z0, z1, z2, gidx, sdst, zrows, *rest):
    outs = rest[:10]
    gidx_v, sdst_v, rows_v, zeros_v, spmem, gsem = rest[10:]
    ztabs = (z0, z1, z2)
    cid = lax.axis_index("c")
    sid = lax.axis_index("s")
    pltpu.sync_copy(zrows, zeros_v)

    for core in range(_NC):
        for ji, (e, row0, rpt) in enumerate(_JOBS[core]):
            n_d = _N_TYPES[_ETYPES[e][1]]
            rows = _agg_rows(n_d)
            q = rows // _NS
            out = outs[_JOB_OUT[core][ji]]

            @pl.when(cid == core)
            def _(e=e, row0=row0, rpt=rpt, q=q, out=out):
                for i in range(q // _ZCH):
                    pltpu.sync_copy(
                        zeros_v, spmem.at[pl.ds(sid * q + i * _ZCH, _ZCH)])
                plsc.subcore_barrier()
                pltpu.sync_copy(gidx.at[e, sid, pl.ds(row0, rpt)],
                                gidx_v.at[pl.ds(0, rpt)])
                pltpu.sync_copy(sdst.at[e, sid, pl.ds(row0, rpt)],
                                sdst_v.at[pl.ds(0, rpt)])
                ztab = ztabs[_ETYPES[e][0]]

                def chunk(j, carry):
                    pltpu.async_copy(ztab.at[gidx_v.at[j]], rows_v, gsem).wait()
                    pltpu.sync_copy(rows_v, spmem.at[sdst_v.at[j]], add=True)
                    return carry

                lax.fori_loop(0, rpt, chunk, 0, unroll=False)
                plsc.subcore_barrier()
                pltpu.sync_copy(spmem.at[pl.ds(sid * q, q)],
                                out.at[pl.ds(sid * q, q)])
                plsc.subcore_barrier()


def _sc_feat_scatter(z0, z1, z2, gidx, sdst, zrows):
    out_type = []
    for core in range(_NC):
        for e, _, _ in _JOBS[core]:
            n_d = _N_TYPES[_ETYPES[e][1]]
            out_type.append(
                jax.ShapeDtypeStruct((_agg_rows(n_d), _D_HID), jnp.float32))
    mesh = plsc.VectorSubcoreMesh(
        core_axis_name="c", subcore_axis_name="s", num_cores=_NC,
        num_subcores=_NS)
    max_rows = max(_agg_rows(n) for n in _N_TYPES)
    call = pl.kernel(
        _feat_scatter_body,
        out_type=tuple(out_type),
        mesh=mesh,
        scratch_types=[
            pltpu.VMEM((_RPT, _CH), jnp.int32),
            pltpu.VMEM((_RPT, _CH), jnp.int32),
            pltpu.VMEM((_CH, _D_HID), jnp.float32),
            pltpu.VMEM((_ZCH, _D_HID), jnp.float32),
            pltpu.VMEM_SHARED((max_rows, _D_HID), jnp.float32),
            pltpu.SemaphoreType.DMA,
        ],
        compiler_params=pltpu.CompilerParams(use_tc_tiling_on_sc=False),
    )
    return call(z0, z1, z2, gidx, sdst, zrows)


def _deg_body(degsrc, degdst, ones, zrows8, *rest):
    outs = rest[:18]
    idx_v, ones_v, zeros_v, spmem = rest[18:]
    cid = lax.axis_index("c")
    sid = lax.axis_index("s")
    pltpu.sync_copy(ones, ones_v)
    pltpu.sync_copy(zrows8, zeros_v)

    for core in range(_NC):
        idx_hbm = degsrc if core == 0 else degdst

        @pl.when(cid == core)
        def _(core=core, idx_hbm=idx_hbm):
            for e in range(9):
                s, d = _ETYPES[e]
                n = _N_TYPES[s] if core == 0 else _N_TYPES[d]
                rows = _deg_rows(n)
                q = rows // _NS
                out = outs[core * 9 + e]
                pltpu.sync_copy(zeros_v.at[pl.ds(0, q)],
                                spmem.at[pl.ds(sid * q, q)])
                plsc.subcore_barrier()
                pltpu.sync_copy(idx_hbm.at[e, sid], idx_v)

                def chunk(j, carry):
                    pltpu.sync_copy(ones_v, spmem.at[idx_v.at[j]], add=True)
                    return carry

                lax.fori_loop(0, _RPT, chunk, 0, unroll=False)
                plsc.subcore_barrier()
                pltpu.sync_copy(spmem.at[pl.ds(sid * q, q)],
                                out.at[pl.ds(sid * q, q)])
                plsc.subcore_barrier()


def _sc_degrees(degsrc, degdst, ones, zrows8):
    out_type = []
    for core in range(_NC):
        for e in range(9):
            s, d = _ETYPES[e]
            n = _N_TYPES[s] if core == 0 else _N_TYPES[d]
            out_type.append(jax.ShapeDtypeStruct((_deg_rows(n), 8), jnp.float32))
    mesh = plsc.VectorSubcoreMesh(
        core_axis_name="c", subcore_axis_name="s", num_cores=_NC,
        num_subcores=_NS)
    max_q = max(_deg_rows(n) for n in _N_TYPES) // _NS
    call = pl.kernel(
        _deg_body,
        out_type=tuple(out_type),
        mesh=mesh,
        scratch_types=[
            pltpu.VMEM((_RPT, _CH), jnp.int32),
            pltpu.VMEM((_CH, 8), jnp.float32),
            pltpu.VMEM((max_q, 8), jnp.float32),
            pltpu.VMEM_SHARED((max(_deg_rows(n) for n in _N_TYPES), 8),
                              jnp.float32),
        ],
        compiler_params=pltpu.CompilerParams(use_tc_tiling_on_sc=False),
    )
    return call(degsrc, degdst, ones, zrows8)


_B = 1000


def _stage_a_body(nk, emb, x_ref, *rest):
    if emb:
        wemb_ref, bemb_ref = rest[0], rest[1]
        rest = rest[2:]
    wcat_ref = rest[0]
    cnt_refs = rest[1:1 + nk]
    out_ref = rest[1 + nk]
    h = x_ref[...]
    if emb:
        h = jnp.dot(h, wemb_ref[...], preferred_element_type=jnp.float32)
        h = h + bemb_ref[...]
    t = jnp.dot(h, wcat_ref[...], preferred_element_type=jnp.float32)
    for k in range(nk):
        s = lax.rsqrt(jnp.maximum(cnt_refs[k][...][:, :1], 1.0))
        out_ref[k] = t[:, _D_HID * k:_D_HID * (k + 1)] * s


def _stage_a(x, wcat, cnts, wemb=None, bemb=None):
    n, din = x.shape
    nk = len(cnts)
    emb = wemb is not None
    in_specs = [pl.BlockSpec((_B, din), lambda i: (i, 0))]
    args = [x]
    if emb:
        in_specs += [pl.BlockSpec((_D_IN, _D_HID), lambda i: (0, 0)),
                     pl.BlockSpec((1, _D_HID), lambda i: (0, 0))]
        args += [wemb, bemb]
    in_specs.append(
        pl.BlockSpec((_D_HID, _D_HID * nk), lambda i: (0, 0)))
    args.append(wcat)
    for c in cnts:
        in_specs.append(pl.BlockSpec((_B, 8), lambda i: (i, 0)))
        args.append(c)
    return pl.pallas_call(
        functools.partial(_stage_a_body, nk, emb),
        grid=(n // _B,),
        in_specs=in_specs,
        out_specs=pl.BlockSpec((nk, _B, _D_HID), lambda i: (0, i, 0)),
        out_shape=jax.ShapeDtypeStruct((nk, n, _D_HID), jnp.float32),
    )(*args)


def _stage_c_body(ncontrib, bias_rows, cls, *refs):
    agg_refs = refs[:ncontrib]
    cnt_refs = refs[ncontrib:2 * ncontrib]
    k = 2 * ncontrib
    bc_ref, g_ref, b_ref = refs[k], refs[k + 1], refs[k + 2]
    k += 3
    if cls:
        wcls_ref, bcls_ref = refs[k], refs[k + 1]
        k += 2
    out_ref = refs[k]
    acc = jnp.zeros((_B, _D_HID), jnp.float32)
    for i in range(ncontrib):
        s = lax.rsqrt(jnp.maximum(cnt_refs[i][...][:, :1], 1.0))
        acc = acc + agg_refs[i][...] * s
    bias = jnp.zeros((1, _D_HID), jnp.float32)
    for r in bias_rows:
        bias = bias + bc_ref[r:r + 1, :]
    acc = acc + bias
    mu = jnp.mean(acc, axis=-1, keepdims=True)
    var = jnp.mean((acc - mu) ** 2, axis=-1, keepdims=True)
    y = (acc - mu) * lax.rsqrt(var + 1e-5) * g_ref[...] + b_ref[...]
    y = jnp.maximum(y, 0.0)
    if cls:
        y = jnp.dot(y, wcls_ref[...], preferred_element_type=jnp.float32)
        y = y + bcls_ref[...]
    out_ref[...] = y


def _stage_c(n, aggs, cnts, bc_l, bias_rows, g, b, wcls=None, bcls=None):
    ncontrib = len(aggs)
    cls = wcls is not None
    in_specs = []
    args = []
    for a in aggs:
        in_specs.append(pl.BlockSpec((_B, _D_HID), lambda i: (i, 0)))
        args.append(a)
    for c in cnts:
        in_specs.append(pl.BlockSpec((_B, 8), lambda i: (i, 0)))
        args.append(c)
    in_specs += [pl.BlockSpec((9, _D_HID), lambda i: (0, 0)),
                 pl.BlockSpec((1, _D_HID), lambda i: (0, 0)),
                 pl.BlockSpec((1, _D_HID), lambda i: (0, 0))]
    args += [bc_l, g, b]
    d_out = _D_HID
    if cls:
        d_out = _D_OUT
        in_specs += [pl.BlockSpec((_D_HID, _D_OUT), lambda i: (0, 0)),
                     pl.BlockSpec((1, _D_OUT), lambda i: (0, 0))]
        args += [wcls, bcls]
    return pl.pallas_call(
        functools.partial(_stage_c_body, ncontrib, tuple(bias_rows), cls),
        grid=(n // _B,),
        in_specs=in_specs,
        out_specs=pl.BlockSpec((_B, d_out), lambda i: (i, 0)),
        out_shape=jax.ShapeDtypeStruct((n, d_out), jnp.float32),
    )(*args)


def kernel(assmpt_feat, rule_feat, non_assmpt_feat, W_emb, b_emb, Wc, bc,
           ln_g, ln_b, Wcls, bcls, edges_src, edges_dst):
    feats = (assmpt_feat, rule_feat, non_assmpt_feat)
    npad = _EPAD - _E

    gidx_l, degsrc_l, sdst_l = [], [], []
    for e, (s, d) in enumerate(_ETYPES):
        n_s, n_d = _N_TYPES[s], _N_TYPES[d]
        src_e, dst_e = edges_src[e], edges_dst[e]
        gidx_l.append(jnp.concatenate(
            [src_e + _SLAB[e] * n_s,
             jnp.full((npad,), _SLAB[e] * n_s, jnp.int32)]))
        degsrc_l.append(jnp.concatenate(
            [src_e, jnp.full((npad,), n_s, jnp.int32)]))
        sdst_l.append(jnp.concatenate(
            [dst_e, jnp.full((npad,), n_d, jnp.int32)]))
    gidx = jnp.stack(gidx_l).reshape(9, _NS, _RPT, _CH)
    degsrc = jnp.stack(degsrc_l).reshape(9, _NS, _RPT, _CH)
    sdst = jnp.stack(sdst_l).reshape(9, _NS, _RPT, _CH)
    ones8 = jnp.ones((_CH, 8), jnp.float32)
    max_q8 = max(_deg_rows(n) for n in _N_TYPES) // _NS
    zrows8 = jnp.zeros((max_q8, 8), jnp.float32)
    zrows = jnp.zeros((_ZCH, _D_HID), jnp.float32)

    degs = _sc_degrees(degsrc, sdst, ones8, zrows8)
    outdeg = degs[:9]
    indeg = degs[9:]

    h = list(feats)
    for l in range(_N_LAYERS):
        ztabs = []
        for s in range(3):
            grp = _SRC_GROUPS[s]
            n_s = _N_TYPES[s]
            wcat = jnp.concatenate([Wc[l, e] for e in grp], axis=1)
            cnts = [outdeg[e][:n_s] for e in grp]
            if l == 0:
                z = _stage_a(h[s], wcat, cnts, wemb=W_emb[s],
                             bemb=b_emb[s][None, :])
            else:
                z = _stage_a(h[s], wcat, cnts)
            ztabs.append(z.reshape(3 * n_s, _D_HID))

        aggs = _sc_feat_scatter(ztabs[0], ztabs[1], ztabs[2], gidx, sdst, zrows)

        h_new = []
        for d in range(3):
            n_d = _N_TYPES[d]
            contribs = _DST_CONTRIBS[d]
            a_views = [aggs[j][:n_d] for j, _ in contribs]
            c_views = [indeg[e][:n_d] for _, e in contribs]
            last = l == _N_LAYERS - 1
            h_new.append(_stage_c(
                n_d, a_views, c_views, bc[l], _DST_GROUPS[d],
                ln_g[d][None, :], ln_b[d][None, :],
                wcls=Wcls[d] if last else None,
                bcls=bcls[d][None, :] if last else None))
        h = h_new

    return h[0], h[1], h[2]


# speedup vs baseline: 6.5457x; 1.1882x over previous
import functools

import jax
import jax.numpy as jnp
from jax import lax
from jax.experimental import pallas as pl
from jax.experimental.pallas import tpu as pltpu
from jax.experimental.pallas import tpu_sc as plsc

_N_TYPES = (20000, 15000, 15000)
_ETYPES = ((0, 1), (2, 1), (2, 0), (0, 0), (1, 2), (1, 0), (0, 0), (1, 1), (2, 2))
_E = 90000
_D_IN, _D_HID, _D_OUT, _N_LAYERS = 128, 64, 8, 3

_NC, _NS = 2, 16
_CH = 128
_EPAD = 90112
_NROW = _EPAD // _CH
_RPT = _NROW // _NS

_SRC_GROUPS = ((0, 3, 6), (4, 5, 7), (1, 2, 8))
_SLAB = {e: k for grp in _SRC_GROUPS for k, e in enumerate(grp)}
_DST_GROUPS = ((2, 3, 5, 6), (0, 1, 7), (4, 8))

_JOBS = (
    ((0, 0, 44), (2, 0, 44), (4, 0, 44), (6, 0, 44), (8, 0, 24)),
    ((1, 0, 44), (3, 0, 44), (5, 0, 44), (7, 0, 44), (8, 24, 20)),
)
_JOB_OUT = ((0, 1, 2, 3, 4), (5, 6, 7, 8, 9))
_DST_CONTRIBS = (
    ((1, 2), (3, 6), (6, 3), (7, 5)),
    ((0, 0), (5, 1), (8, 7)),
    ((2, 4), (4, 8), (9, 8)),
)

_ZCH = 24


def _agg_rows(n):
    q = -(-(n + 1) // _NS)
    q = -(-q // _ZCH) * _ZCH
    return _NS * q


def _deg_rows(n):
    q = -(-(n + 1) // _NS)
    q = -(-q // 128) * 128
    return _NS * q


_RCH = 2


def _feat_scatter_body(z0, z1, z2, gidx, sdst, zrows, *rest):
    outs = rest[:10]
    gidx_v, sdst_v, rows0_v, rows1_v, zeros_v, spmem, sem0, sem1 = rest[10:]
    ztabs = (z0, z1, z2)
    rows_v = (rows0_v, rows1_v)
    sems = (sem0, sem1)
    cid = lax.axis_index("c")
    sid = lax.axis_index("s")
    pltpu.sync_copy(zrows, zeros_v)

    for core in range(_NC):
        for ji, (e, row0, rpt) in enumerate(_JOBS[core]):
            n_d = _N_TYPES[_ETYPES[e][1]]
            rows = _agg_rows(n_d)
            q = rows // _NS
            out = outs[_JOB_OUT[core][ji]]

            @pl.when(cid == core)
            def _(e=e, row0=row0, rpt=rpt, q=q, out=out):
                for i in range(q // _ZCH):
                    pltpu.sync_copy(
                        zeros_v, spmem.at[pl.ds(sid * q + i * _ZCH, _ZCH)])
                plsc.subcore_barrier()
                ne = rpt * _CH
                pltpu.sync_copy(gidx.at[e, sid, pl.ds(row0 * _CH, ne)],
                                gidx_v.at[pl.ds(0, ne)])
                pltpu.sync_copy(sdst.at[e, sid, pl.ds(row0 * _CH, ne)],
                                sdst_v.at[pl.ds(0, ne)])
                ztab = ztabs[_ETYPES[e][0]]

                ec = _RCH * _CH
                nch = rpt // _RCH
                descs = [None, None]
                descs[0] = pltpu.async_copy(
                    ztab.at[gidx_v.at[pl.ds(0, ec)]], rows_v[0], sems[0])
                for ci in range(nch):
                    b = ci % 2
                    descs[b].wait()
                    if ci + 1 < nch:
                        nb = (ci + 1) % 2
                        descs[nb] = pltpu.async_copy(
                            ztab.at[gidx_v.at[pl.ds((ci + 1) * ec, ec)]],
                            rows_v[nb], sems[nb])
                    pltpu.sync_copy(
                        rows_v[b],
                        spmem.at[sdst_v.at[pl.ds(ci * ec, ec)]], add=True)

                plsc.subcore_barrier()
                pltpu.sync_copy(spmem.at[pl.ds(sid * q, q)],
                                out.at[pl.ds(sid * q, q)])
                plsc.subcore_barrier()


def _sc_feat_scatter(z0, z1, z2, gidx, sdst, zrows):
    out_type = []
    for core in range(_NC):
        for e, _, _ in _JOBS[core]:
            n_d = _N_TYPES[_ETYPES[e][1]]
            out_type.append(
                jax.ShapeDtypeStruct((_agg_rows(n_d), _D_HID), jnp.float32))
    mesh = plsc.VectorSubcoreMesh(
        core_axis_name="c", subcore_axis_name="s", num_cores=_NC,
        num_subcores=_NS)
    max_rows = max(_agg_rows(n) for n in _N_TYPES)
    call = pl.kernel(
        _feat_scatter_body,
        out_type=tuple(out_type),
        mesh=mesh,
        scratch_types=[
            pltpu.VMEM((_RPT * _CH,), jnp.int32),
            pltpu.VMEM((_RPT * _CH,), jnp.int32),
            pltpu.VMEM((_RCH * _CH, _D_HID), jnp.float32),
            pltpu.VMEM((_RCH * _CH, _D_HID), jnp.float32),
            pltpu.VMEM((_ZCH, _D_HID), jnp.float32),
            pltpu.VMEM_SHARED((max_rows, _D_HID), jnp.float32),
            pltpu.SemaphoreType.DMA,
            pltpu.SemaphoreType.DMA,
        ],
        compiler_params=pltpu.CompilerParams(use_tc_tiling_on_sc=False),
    )
    return call(z0, z1, z2, gidx, sdst, zrows)


def _deg_body(degsrc, degdst, ones, zrows8, *rest):
    outs = rest[:18]
    idx_v, ones_v, zeros_v, spmem = rest[18:]
    cid = lax.axis_index("c")
    sid = lax.axis_index("s")
    pltpu.sync_copy(ones, ones_v)
    pltpu.sync_copy(zrows8, zeros_v)

    for core in range(_NC):
        idx_hbm = degsrc if core == 0 else degdst

        @pl.when(cid == core)
        def _(core=core, idx_hbm=idx_hbm):
            for e in range(9):
                s, d = _ETYPES[e]
                n = _N_TYPES[s] if core == 0 else _N_TYPES[d]
                rows = _deg_rows(n)
                q = rows // _NS
                out = outs[core * 9 + e]
                pltpu.sync_copy(zeros_v.at[pl.ds(0, q)],
                                spmem.at[pl.ds(sid * q, q)])
                plsc.subcore_barrier()
                pltpu.sync_copy(idx_hbm.at[e, sid], idx_v)
                ec = _RCH * _CH

                def chunk(j, carry):
                    pltpu.sync_copy(
                        ones_v, spmem.at[idx_v.at[pl.ds(j * ec, ec)]],
                        add=True)
                    return carry

                lax.fori_loop(0, _RPT // _RCH, chunk, 0, unroll=False)
                plsc.subcore_barrier()
                pltpu.sync_copy(spmem.at[pl.ds(sid * q, q)],
                                out.at[pl.ds(sid * q, q)])
                plsc.subcore_barrier()


def _sc_degrees(degsrc, degdst, ones, zrows8):
    out_type = []
    for core in range(_NC):
        for e in range(9):
            s, d = _ETYPES[e]
            n = _N_TYPES[s] if core == 0 else _N_TYPES[d]
            out_type.append(jax.ShapeDtypeStruct((_deg_rows(n), 8), jnp.float32))
    mesh = plsc.VectorSubcoreMesh(
        core_axis_name="c", subcore_axis_name="s", num_cores=_NC,
        num_subcores=_NS)
    max_q = max(_deg_rows(n) for n in _N_TYPES) // _NS
    call = pl.kernel(
        _deg_body,
        out_type=tuple(out_type),
        mesh=mesh,
        scratch_types=[
            pltpu.VMEM((_RPT * _CH,), jnp.int32),
            pltpu.VMEM((_RCH * _CH, 8), jnp.float32),
            pltpu.VMEM((max_q, 8), jnp.float32),
            pltpu.VMEM_SHARED((max(_deg_rows(n) for n in _N_TYPES), 8),
                              jnp.float32),
        ],
        compiler_params=pltpu.CompilerParams(use_tc_tiling_on_sc=False),
    )
    return call(degsrc, degdst, ones, zrows8)


_B = 1000


def _stage_a_body(nk, emb, x_ref, *rest):
    if emb:
        wemb_ref, bemb_ref = rest[0], rest[1]
        rest = rest[2:]
    wcat_ref = rest[0]
    cnt_refs = rest[1:1 + nk]
    out_ref = rest[1 + nk]
    h = x_ref[...]
    if emb:
        h = jnp.dot(h, wemb_ref[...], preferred_element_type=jnp.float32)
        h = h + bemb_ref[...]
    t = jnp.dot(h, wcat_ref[...], preferred_element_type=jnp.float32)
    for k in range(nk):
        s = lax.rsqrt(jnp.maximum(cnt_refs[k][...][:, :1], 1.0))
        out_ref[k] = t[:, _D_HID * k:_D_HID * (k + 1)] * s


def _stage_a(x, wcat, cnts, wemb=None, bemb=None):
    n, din = x.shape
    nk = len(cnts)
    emb = wemb is not None
    in_specs = [pl.BlockSpec((_B, din), lambda i: (i, 0))]
    args = [x]
    if emb:
        in_specs += [pl.BlockSpec((_D_IN, _D_HID), lambda i: (0, 0)),
                     pl.BlockSpec((1, _D_HID), lambda i: (0, 0))]
        args += [wemb, bemb]
    in_specs.append(
        pl.BlockSpec((_D_HID, _D_HID * nk), lambda i: (0, 0)))
    args.append(wcat)
    for c in cnts:
        in_specs.append(pl.BlockSpec((_B, 8), lambda i: (i, 0)))
        args.append(c)
    return pl.pallas_call(
        functools.partial(_stage_a_body, nk, emb),
        grid=(n // _B,),
        in_specs=in_specs,
        out_specs=pl.BlockSpec((nk, _B, _D_HID), lambda i: (0, i, 0)),
        out_shape=jax.ShapeDtypeStruct((nk, n, _D_HID), jnp.float32),
    )(*args)


def _stage_c_body(ncontrib, bias_rows, cls, *refs):
    agg_refs = refs[:ncontrib]
    cnt_refs = refs[ncontrib:2 * ncontrib]
    k = 2 * ncontrib
    bc_ref, g_ref, b_ref = refs[k], refs[k + 1], refs[k + 2]
    k += 3
    if cls:
        wcls_ref, bcls_ref = refs[k], refs[k + 1]
        k += 2
    out_ref = refs[k]
    acc = jnp.zeros((_B, _D_HID), jnp.float32)
    for i in range(ncontrib):
        s = lax.rsqrt(jnp.maximum(cnt_refs[i][...][:, :1], 1.0))
        acc = acc + agg_refs[i][...] * s
    bias = jnp.zeros((1, _D_HID), jnp.float32)
    for r in bias_rows:
        bias = bias + bc_ref[r:r + 1, :]
    acc = acc + bias
    mu = jnp.mean(acc, axis=-1, keepdims=True)
    var = jnp.mean((acc - mu) ** 2, axis=-1, keepdims=True)
    y = (acc - mu) * lax.rsqrt(var + 1e-5) * g_ref[...] + b_ref[...]
    y = jnp.maximum(y, 0.0)
    if cls:
        y = jnp.dot(y, wcls_ref[...], preferred_element_type=jnp.float32)
        y = y + bcls_ref[...]
    out_ref[...] = y


def _stage_c(n, aggs, cnts, bc_l, bias_rows, g, b, wcls=None, bcls=None):
    ncontrib = len(aggs)
    cls = wcls is not None
    in_specs = []
    args = []
    for a in aggs:
        in_specs.append(pl.BlockSpec((_B, _D_HID), lambda i: (i, 0)))
        args.append(a)
    for c in cnts:
        in_specs.append(pl.BlockSpec((_B, 8), lambda i: (i, 0)))
        args.append(c)
    in_specs += [pl.BlockSpec((9, _D_HID), lambda i: (0, 0)),
                 pl.BlockSpec((1, _D_HID), lambda i: (0, 0)),
                 pl.BlockSpec((1, _D_HID), lambda i: (0, 0))]
    args += [bc_l, g, b]
    d_out = _D_HID
    if cls:
        d_out = _D_OUT
        in_specs += [pl.BlockSpec((_D_HID, _D_OUT), lambda i: (0, 0)),
                     pl.BlockSpec((1, _D_OUT), lambda i: (0, 0))]
        args += [wcls, bcls]
    return pl.pallas_call(
        functools.partial(_stage_c_body, ncontrib, tuple(bias_rows), cls),
        grid=(n // _B,),
        in_specs=in_specs,
        out_specs=pl.BlockSpec((_B, d_out), lambda i: (i, 0)),
        out_shape=jax.ShapeDtypeStruct((n, d_out), jnp.float32),
    )(*args)


def kernel(assmpt_feat, rule_feat, non_assmpt_feat, W_emb, b_emb, Wc, bc,
           ln_g, ln_b, Wcls, bcls, edges_src, edges_dst):
    feats = (assmpt_feat, rule_feat, non_assmpt_feat)
    npad = _EPAD - _E

    gidx_l, degsrc_l, sdst_l = [], [], []
    for e, (s, d) in enumerate(_ETYPES):
        n_s, n_d = _N_TYPES[s], _N_TYPES[d]
        src_e, dst_e = edges_src[e], edges_dst[e]
        gidx_l.append(jnp.concatenate(
            [src_e + _SLAB[e] * n_s,
             jnp.full((npad,), _SLAB[e] * n_s, jnp.int32)]))
        degsrc_l.append(jnp.concatenate(
            [src_e, jnp.full((npad,), n_s, jnp.int32)]))
        sdst_l.append(jnp.concatenate(
            [dst_e, jnp.full((npad,), n_d, jnp.int32)]))
    gidx = jnp.stack(gidx_l).reshape(9, _NS, _RPT * _CH)
    degsrc = jnp.stack(degsrc_l).reshape(9, _NS, _RPT * _CH)
    sdst = jnp.stack(sdst_l).reshape(9, _NS, _RPT * _CH)
    ones8 = jnp.ones((_RCH * _CH, 8), jnp.float32)
    max_q8 = max(_deg_rows(n) for n in _N_TYPES) // _NS
    zrows8 = jnp.zeros((max_q8, 8), jnp.float32)
    zrows = jnp.zeros((_ZCH, _D_HID), jnp.float32)

    degs = _sc_degrees(degsrc, sdst, ones8, zrows8)
    outdeg = degs[:9]
    indeg = degs[9:]

    h = list(feats)
    for l in range(_N_LAYERS):
        ztabs = []
        for s in range(3):
            grp = _SRC_GROUPS[s]
            n_s = _N_TYPES[s]
            wcat = jnp.concatenate([Wc[l, e] for e in grp], axis=1)
            cnts = [outdeg[e][:n_s] for e in grp]
            if l == 0:
                z = _stage_a(h[s], wcat, cnts, wemb=W_emb[s],
                             bemb=b_emb[s][None, :])
            else:
                z = _stage_a(h[s], wcat, cnts)
            ztabs.append(z.reshape(3 * n_s, _D_HID))

        aggs = _sc_feat_scatter(ztabs[0], ztabs[1], ztabs[2], gidx, sdst, zrows)

        h_new = []
        for d in range(3):
            n_d = _N_TYPES[d]
            contribs = _DST_CONTRIBS[d]
            a_views = [aggs[j][:n_d] for j, _ in contribs]
            c_views = [indeg[e][:n_d] for _, e in contribs]
            last = l == _N_LAYERS - 1
            h_new.append(_stage_c(
                n_d, a_views, c_views, bc[l], _DST_GROUPS[d],
                ln_g[d][None, :], ln_b[d][None, :],
                wcls=Wcls[d] if last else None,
                bcls=bcls[d][None, :] if last else None))
        h = h_new

    return h[0], h[1], h[2]


# speedup vs baseline: 6.8631x; 1.0485x over previous
import functools

import jax
import jax.numpy as jnp
from jax import lax
from jax.experimental import pallas as pl
from jax.experimental.pallas import tpu as pltpu
from jax.experimental.pallas import tpu_sc as plsc

_N_TYPES = (20000, 15000, 15000)
_ETYPES = ((0, 1), (2, 1), (2, 0), (0, 0), (1, 2), (1, 0), (0, 0), (1, 1), (2, 2))
_E = 90000
_D_IN, _D_HID, _D_OUT, _N_LAYERS = 128, 64, 8, 3

_NC, _NS = 2, 16
_CH = 128
_EPAD = 90112
_NROW = _EPAD // _CH
_RPT = _NROW // _NS

_SRC_GROUPS = ((0, 3, 6), (4, 5, 7), (1, 2, 8))
_SLAB = {e: k for grp in _SRC_GROUPS for k, e in enumerate(grp)}
_DST_GROUPS = ((2, 3, 5, 6), (0, 1, 7), (4, 8))

_JOBS = (
    ((0, 0, 44), (2, 0, 44), (4, 0, 44), (6, 0, 44), (8, 0, 24)),
    ((1, 0, 44), (3, 0, 44), (5, 0, 44), (7, 0, 44), (8, 24, 20)),
)
_JOB_OUT = ((0, 1, 2, 3, 4), (5, 6, 7, 8, 9))
_DST_CONTRIBS = (
    ((1, 2), (3, 6), (6, 3), (7, 5)),
    ((0, 0), (5, 1), (8, 7)),
    ((2, 4), (4, 8), (9, 8)),
)

_ZCH = 24


def _agg_rows(n):
    q = -(-(n + 1) // _NS)
    q = -(-q // _ZCH) * _ZCH
    return _NS * q


def _deg_rows(n):
    q = -(-(n + 1) // _NS)
    q = -(-q // 128) * 128
    return _NS * q


_RCH = 2


def _feat_scatter_body(*all_args):
    ztabs_flat = all_args[:9]
    gidx, sdst, zrows = all_args[9:12]
    rest = all_args[12:]
    outs = rest[:10]
    gidx_v, sdst_v, rows0_v, rows1_v, zeros_v, spmem, sem0, sem1 = rest[10:]
    ztabs = (ztabs_flat[0:3], ztabs_flat[3:6], ztabs_flat[6:9])
    rows_v = (rows0_v, rows1_v)
    sems = (sem0, sem1)
    cid = lax.axis_index("c")
    sid = lax.axis_index("s")
    pltpu.sync_copy(zrows, zeros_v)

    for core in range(_NC):
        for ji, (e, row0, rpt) in enumerate(_JOBS[core]):
            n_d = _N_TYPES[_ETYPES[e][1]]
            rows = _agg_rows(n_d)
            q = rows // _NS
            out = outs[_JOB_OUT[core][ji]]

            @pl.when(cid == core)
            def _(e=e, row0=row0, rpt=rpt, q=q, out=out):
                for i in range(q // _ZCH):
                    pltpu.sync_copy(
                        zeros_v, spmem.at[pl.ds(sid * q + i * _ZCH, _ZCH)])
                plsc.subcore_barrier()
                ne = rpt * _CH
                pltpu.sync_copy(gidx.at[e, sid, pl.ds(row0 * _CH, ne)],
                                gidx_v.at[pl.ds(0, ne)])
                pltpu.sync_copy(sdst.at[e, sid, pl.ds(row0 * _CH, ne)],
                                sdst_v.at[pl.ds(0, ne)])
                ztab = ztabs[_ETYPES[e][0]][_SLAB[e]]

                ec = _RCH * _CH
                nch = rpt // _RCH
                descs = [None, None]
                descs[0] = pltpu.async_copy(
                    ztab.at[gidx_v.at[pl.ds(0, ec)]], rows_v[0], sems[0])
                for ci in range(nch):
                    b = ci % 2
                    descs[b].wait()
                    if ci + 1 < nch:
                        nb = (ci + 1) % 2
                        descs[nb] = pltpu.async_copy(
                            ztab.at[gidx_v.at[pl.ds((ci + 1) * ec, ec)]],
                            rows_v[nb], sems[nb])
                    pltpu.sync_copy(
                        rows_v[b],
                        spmem.at[sdst_v.at[pl.ds(ci * ec, ec)]], add=True)

                plsc.subcore_barrier()
                pltpu.sync_copy(spmem.at[pl.ds(sid * q, q)],
                                out.at[pl.ds(sid * q, q)])
                plsc.subcore_barrier()


def _sc_feat_scatter(ztabs_flat, gidx, sdst, zrows):
    out_type = []
    for core in range(_NC):
        for e, _, _ in _JOBS[core]:
            n_d = _N_TYPES[_ETYPES[e][1]]
            out_type.append(
                jax.ShapeDtypeStruct((_agg_rows(n_d), _D_HID), jnp.float32))
    mesh = plsc.VectorSubcoreMesh(
        core_axis_name="c", subcore_axis_name="s", num_cores=_NC,
        num_subcores=_NS)
    max_rows = max(_agg_rows(n) for n in _N_TYPES)
    call = pl.kernel(
        _feat_scatter_body,
        out_type=tuple(out_type),
        mesh=mesh,
        scratch_types=[
            pltpu.VMEM((_RPT * _CH,), jnp.int32),
            pltpu.VMEM((_RPT * _CH,), jnp.int32),
            pltpu.VMEM((_RCH * _CH, _D_HID), jnp.float32),
            pltpu.VMEM((_RCH * _CH, _D_HID), jnp.float32),
            pltpu.VMEM((_ZCH, _D_HID), jnp.float32),
            pltpu.VMEM_SHARED((max_rows, _D_HID), jnp.float32),
            pltpu.SemaphoreType.DMA,
            pltpu.SemaphoreType.DMA,
        ],
        compiler_params=pltpu.CompilerParams(use_tc_tiling_on_sc=False),
    )
    return call(*ztabs_flat, gidx, sdst, zrows)


def _deg_body(degsrc, degdst, ones, zrows8, *rest):
    outs = rest[:18]
    idx_v, ones_v, zeros_v, spmem = rest[18:]
    cid = lax.axis_index("c")
    sid = lax.axis_index("s")
    pltpu.sync_copy(ones, ones_v)
    pltpu.sync_copy(zrows8, zeros_v)

    for core in range(_NC):
        idx_hbm = degsrc if core == 0 else degdst

        @pl.when(cid == core)
        def _(core=core, idx_hbm=idx_hbm):
            for e in range(9):
                s, d = _ETYPES[e]
                n = _N_TYPES[s] if core == 0 else _N_TYPES[d]
                rows = _deg_rows(n)
                q = rows // _NS
                out = outs[core * 9 + e]
                pltpu.sync_copy(zeros_v.at[pl.ds(0, q)],
                                spmem.at[pl.ds(sid * q, q)])
                plsc.subcore_barrier()
                pltpu.sync_copy(idx_hbm.at[e, sid], idx_v)
                ec = _RCH * _CH

                def chunk(j, carry):
                    pltpu.sync_copy(
                        ones_v, spmem.at[idx_v.at[pl.ds(j * ec, ec)]],
                        add=True)
                    return carry

                lax.fori_loop(0, _RPT // _RCH, chunk, 0, unroll=False)
                plsc.subcore_barrier()
                pltpu.sync_copy(spmem.at[pl.ds(sid * q, q)],
                                out.at[pl.ds(sid * q, q)])
                plsc.subcore_barrier()


def _sc_degrees(degsrc, degdst, ones, zrows8):
    out_type = []
    for core in range(_NC):
        for e in range(9):
            s, d = _ETYPES[e]
            n = _N_TYPES[s] if core == 0 else _N_TYPES[d]
            out_type.append(jax.ShapeDtypeStruct((_deg_rows(n), 8), jnp.float32))
    mesh = plsc.VectorSubcoreMesh(
        core_axis_name="c", subcore_axis_name="s", num_cores=_NC,
        num_subcores=_NS)
    max_q = max(_deg_rows(n) for n in _N_TYPES) // _NS
    call = pl.kernel(
        _deg_body,
        out_type=tuple(out_type),
        mesh=mesh,
        scratch_types=[
            pltpu.VMEM((_RPT * _CH,), jnp.int32),
            pltpu.VMEM((_RCH * _CH, 8), jnp.float32),
            pltpu.VMEM((max_q, 8), jnp.float32),
            pltpu.VMEM_SHARED((max(_deg_rows(n) for n in _N_TYPES), 8),
                              jnp.float32),
        ],
        compiler_params=pltpu.CompilerParams(use_tc_tiling_on_sc=False),
    )
    return call(degsrc, degdst, ones, zrows8)


_B = 1000


def _a_tail(h, wcat_ref, ocnt_refs, out_refs):
    t = jnp.dot(h, wcat_ref[...], preferred_element_type=jnp.float32)
    for k in range(3):
        s = lax.rsqrt(jnp.maximum(ocnt_refs[k][...][:, :1], 1.0))
        out_refs[k][...] = t[:, _D_HID * k:_D_HID * (k + 1)] * s


def _embed_a_body(x_ref, wemb_ref, bemb_ref, wcat_ref, c0, c1, c2,
                  o0, o1, o2):
    h = jnp.dot(x_ref[...], wemb_ref[...], preferred_element_type=jnp.float32)
    h = h + bemb_ref[...]
    _a_tail(h, wcat_ref, (c0, c1, c2), (o0, o1, o2))


def _z_out(n):
    return (
        [pl.BlockSpec((_B, _D_HID), lambda i: (i, 0))] * 3,
        [jax.ShapeDtypeStruct((n, _D_HID), jnp.float32)] * 3,
    )


def _embed_a(x, wemb, bemb, wcat, ocnts):
    n = x.shape[0]
    in_specs = [pl.BlockSpec((_B, _D_IN), lambda i: (i, 0)),
                pl.BlockSpec((_D_IN, _D_HID), lambda i: (0, 0)),
                pl.BlockSpec((1, _D_HID), lambda i: (0, 0)),
                pl.BlockSpec((_D_HID, 3 * _D_HID), lambda i: (0, 0))]
    in_specs += [pl.BlockSpec((_B, 8), lambda i: (i, 0))] * 3
    out_specs, out_shape = _z_out(n)
    return pl.pallas_call(
        _embed_a_body,
        grid=(n // _B,),
        in_specs=in_specs,
        out_specs=out_specs,
        out_shape=out_shape,
    )(x, wemb, bemb, wcat, *ocnts)


def _c_mid(ncontrib, bias_rows, refs):
    agg_refs = refs[:ncontrib]
    cnt_refs = refs[ncontrib:2 * ncontrib]
    k = 2 * ncontrib
    bc_ref, g_ref, b_ref = refs[k], refs[k + 1], refs[k + 2]
    acc = jnp.zeros((_B, _D_HID), jnp.float32)
    for i in range(ncontrib):
        s = lax.rsqrt(jnp.maximum(cnt_refs[i][...][:, :1], 1.0))
        acc = acc + agg_refs[i][...] * s
    bias = jnp.zeros((1, _D_HID), jnp.float32)
    for r in bias_rows:
        bias = bias + bc_ref[r:r + 1, :]
    acc = acc + bias
    mu = jnp.mean(acc, axis=-1, keepdims=True)
    var = jnp.mean((acc - mu) ** 2, axis=-1, keepdims=True)
    y = (acc - mu) * lax.rsqrt(var + 1e-5) * g_ref[...] + b_ref[...]
    return jnp.maximum(y, 0.0), 2 * ncontrib + 3


def _fused_ca_body(ncontrib, bias_rows, *refs):
    y, k = _c_mid(ncontrib, bias_rows, refs)
    wcat_ref = refs[k]
    ocnt_refs = refs[k + 1:k + 4]
    out_refs = refs[k + 4:k + 7]
    _a_tail(y, wcat_ref, ocnt_refs, out_refs)


def _final_c_body(ncontrib, bias_rows, *refs):
    y, k = _c_mid(ncontrib, bias_rows, refs)
    wcls_ref, bcls_ref, out_ref = refs[k], refs[k + 1], refs[k + 2]
    y = jnp.dot(y, wcls_ref[...], preferred_element_type=jnp.float32)
    out_ref[...] = y + bcls_ref[...]


def _c_specs_args(aggs, cnts, bc_l, g, b):
    in_specs = []
    args = []
    for a in aggs:
        in_specs.append(pl.BlockSpec((_B, _D_HID), lambda i: (i, 0)))
        args.append(a)
    for c in cnts:
        in_specs.append(pl.BlockSpec((_B, 8), lambda i: (i, 0)))
        args.append(c)
    in_specs += [pl.BlockSpec((9, _D_HID), lambda i: (0, 0)),
                 pl.BlockSpec((1, _D_HID), lambda i: (0, 0)),
                 pl.BlockSpec((1, _D_HID), lambda i: (0, 0))]
    args += [bc_l, g, b]
    return in_specs, args


def _fused_ca(n, aggs, cnts, bc_l, bias_rows, g, b, wcat, ocnts):
    in_specs, args = _c_specs_args(aggs, cnts, bc_l, g, b)
    in_specs.append(pl.BlockSpec((_D_HID, 3 * _D_HID), lambda i: (0, 0)))
    args.append(wcat)
    in_specs += [pl.BlockSpec((_B, 8), lambda i: (i, 0))] * 3
    args += list(ocnts)
    out_specs, out_shape = _z_out(n)
    return pl.pallas_call(
        functools.partial(_fused_ca_body, len(aggs), tuple(bias_rows)),
        grid=(n // _B,),
        in_specs=in_specs,
        out_specs=out_specs,
        out_shape=out_shape,
    )(*args)


def _final_c(n, aggs, cnts, bc_l, bias_rows, g, b, wcls, bcls):
    in_specs, args = _c_specs_args(aggs, cnts, bc_l, g, b)
    in_specs += [pl.BlockSpec((_D_HID, _D_OUT), lambda i: (0, 0)),
                 pl.BlockSpec((1, _D_OUT), lambda i: (0, 0))]
    args += [wcls, bcls]
    return pl.pallas_call(
        functools.partial(_final_c_body, len(aggs), tuple(bias_rows)),
        grid=(n // _B,),
        in_specs=in_specs,
        out_specs=pl.BlockSpec((_B, _D_OUT), lambda i: (i, 0)),
        out_shape=jax.ShapeDtypeStruct((n, _D_OUT), jnp.float32),
    )(*args)


def kernel(assmpt_feat, rule_feat, non_assmpt_feat, W_emb, b_emb, Wc, bc,
           ln_g, ln_b, Wcls, bcls, edges_src, edges_dst):
    feats = (assmpt_feat, rule_feat, non_assmpt_feat)
    npad = _EPAD - _E

    gidx_l, degsrc_l, sdst_l = [], [], []
    for e, (s, d) in enumerate(_ETYPES):
        n_s, n_d = _N_TYPES[s], _N_TYPES[d]
        src_e, dst_e = edges_src[e], edges_dst[e]
        gidx_l.append(jnp.concatenate(
            [src_e, jnp.zeros((npad,), jnp.int32)]))
        degsrc_l.append(jnp.concatenate(
            [src_e, jnp.full((npad,), n_s, jnp.int32)]))
        sdst_l.append(jnp.concatenate(
            [dst_e, jnp.full((npad,), n_d, jnp.int32)]))
    gidx = jnp.stack(gidx_l).reshape(9, _NS, _RPT * _CH)
    degsrc = jnp.stack(degsrc_l).reshape(9, _NS, _RPT * _CH)
    sdst = jnp.stack(sdst_l).reshape(9, _NS, _RPT * _CH)
    ones8 = jnp.ones((_RCH * _CH, 8), jnp.float32)
    max_q8 = max(_deg_rows(n) for n in _N_TYPES) // _NS
    zrows8 = jnp.zeros((max_q8, 8), jnp.float32)
    zrows = jnp.zeros((_ZCH, _D_HID), jnp.float32)

    degs = _sc_degrees(degsrc, sdst, ones8, zrows8)
    outdeg = degs[:9]
    indeg = degs[9:]

    def wcat_l(l, s):
        return jnp.concatenate([Wc[l, e] for e in _SRC_GROUPS[s]], axis=1)

    def ocnts_s(s):
        return [outdeg[e][:_N_TYPES[s]] for e in _SRC_GROUPS[s]]

    ztabs = [_embed_a(feats[s], W_emb[s], b_emb[s][None, :], wcat_l(0, s),
                      ocnts_s(s)) for s in range(3)]

    out = [None, None, None]
    for l in range(_N_LAYERS):
        aggs = _sc_feat_scatter(
            [t for zs in ztabs for t in zs], gidx, sdst, zrows)
        last = l == _N_LAYERS - 1
        nxt = []
        for d in range(3):
            n_d = _N_TYPES[d]
            contribs = _DST_CONTRIBS[d]
            a_views = [aggs[j][:n_d] for j, _ in contribs]
            c_views = [indeg[e][:n_d] for _, e in contribs]
            if last:
                out[d] = _final_c(
                    n_d, a_views, c_views, bc[l], _DST_GROUPS[d],
                    ln_g[d][None, :], ln_b[d][None, :],
                    Wcls[d], bcls[d][None, :])
            else:
                nxt.append(_fused_ca(
                    n_d, a_views, c_views, bc[l], _DST_GROUPS[d],
                    ln_g[d][None, :], ln_b[d][None, :],
                    wcat_l(l + 1, d), ocnts_s(d)))
        ztabs = nxt

    return out[0], out[1], out[2]


# speedup vs baseline: 7.9151x; 1.1533x over previous
import functools

import jax
import jax.numpy as jnp
from jax import lax
from jax.experimental import pallas as pl
from jax.experimental.pallas import tpu as pltpu
from jax.experimental.pallas import tpu_sc as plsc

_N_TYPES = (20000, 15000, 15000)
_ETYPES = ((0, 1), (2, 1), (2, 0), (0, 0), (1, 2), (1, 0), (0, 0), (1, 1), (2, 2))
_E = 90000
_D_IN, _D_HID, _D_OUT, _N_LAYERS = 128, 64, 8, 3

_NC, _NS = 2, 16
_CH = 128
_EPAD = 90112
_NROW = _EPAD // _CH
_RPT = _NROW // _NS

_SRC_GROUPS = ((0, 3, 6), (4, 5, 7), (1, 2, 8))
_SLAB = {e: k for grp in _SRC_GROUPS for k, e in enumerate(grp)}
_DST_GROUPS = ((2, 3, 5, 6), (0, 1, 7), (4, 8))

_JOBS = (
    ((0, 0, 44), (2, 0, 44), (4, 0, 44), (6, 0, 44), (8, 0, 24)),
    ((1, 0, 44), (3, 0, 44), (5, 0, 44), (7, 0, 44), (8, 24, 20)),
)
_JOB_OUT = ((0, 1, 2, 3, 4), (5, 6, 7, 8, 9))
_DST_CONTRIBS = (
    ((1, 2), (3, 6), (6, 3), (7, 5)),
    ((0, 0), (5, 1), (8, 7)),
    ((2, 4), (4, 8), (9, 8)),
)

_ZCH = 24


def _agg_rows(n):
    q = -(-(n + 1) // _NS)
    q = -(-q // _ZCH) * _ZCH
    return _NS * q


def _deg_rows(n):
    q = -(-(n + 1) // _NS)
    q = -(-q // 128) * 128
    return _NS * q


_RCH = 2


def _feat_scatter_body(*all_args):
    ztabs_flat = all_args[:9]
    gidx, sdst, zrows = all_args[9:12]
    rest = all_args[12:]
    outs = rest[:10]
    gidx_v, sdst_v, rows0_v, rows1_v, zeros_v, spmem, sem0, sem1 = rest[10:]
    ztabs = (ztabs_flat[0:3], ztabs_flat[3:6], ztabs_flat[6:9])
    rows_v = (rows0_v, rows1_v)
    sems = (sem0, sem1)
    cid = lax.axis_index("c")
    sid = lax.axis_index("s")
    pltpu.sync_copy(zrows, zeros_v)

    for core in range(_NC):
        for ji, (e, row0, rpt) in enumerate(_JOBS[core]):
            n_d = _N_TYPES[_ETYPES[e][1]]
            rows = _agg_rows(n_d)
            q = rows // _NS
            out = outs[_JOB_OUT[core][ji]]

            @pl.when(cid == core)
            def _(e=e, row0=row0, rpt=rpt, q=q, out=out):
                for i in range(q // _ZCH):
                    pltpu.sync_copy(
                        zeros_v, spmem.at[pl.ds(sid * q + i * _ZCH, _ZCH)])
                plsc.subcore_barrier()
                ne = rpt * _CH
                pltpu.sync_copy(gidx.at[e, sid, pl.ds(row0 * _CH, ne)],
                                gidx_v.at[pl.ds(0, ne)])
                pltpu.sync_copy(sdst.at[e, sid, pl.ds(row0 * _CH, ne)],
                                sdst_v.at[pl.ds(0, ne)])
                ztab = ztabs[_ETYPES[e][0]][_SLAB[e]]

                ec = _RCH * _CH
                nch = rpt // _RCH
                descs = [None, None]
                descs[0] = pltpu.async_copy(
                    ztab.at[gidx_v.at[pl.ds(0, ec)]], rows_v[0], sems[0])
                for ci in range(nch):
                    b = ci % 2
                    descs[b].wait()
                    if ci + 1 < nch:
                        nb = (ci + 1) % 2
                        descs[nb] = pltpu.async_copy(
                            ztab.at[gidx_v.at[pl.ds((ci + 1) * ec, ec)]],
                            rows_v[nb], sems[nb])
                    pltpu.sync_copy(
                        rows_v[b],
                        spmem.at[sdst_v.at[pl.ds(ci * ec, ec)]], add=True)

                plsc.subcore_barrier()
                pltpu.sync_copy(spmem.at[pl.ds(sid * q, q)],
                                out.at[pl.ds(sid * q, q)])
                plsc.subcore_barrier()


def _sc_feat_scatter(ztabs_flat, gidx, sdst, zrows):
    out_type = []
    for core in range(_NC):
        for e, _, _ in _JOBS[core]:
            n_d = _N_TYPES[_ETYPES[e][1]]
            out_type.append(
                jax.ShapeDtypeStruct((_agg_rows(n_d), _D_HID), jnp.float32))
    mesh = plsc.VectorSubcoreMesh(
        core_axis_name="c", subcore_axis_name="s", num_cores=_NC,
        num_subcores=_NS)
    max_rows = max(_agg_rows(n) for n in _N_TYPES)
    call = pl.kernel(
        _feat_scatter_body,
        out_type=tuple(out_type),
        mesh=mesh,
        scratch_types=[
            pltpu.VMEM((_RPT * _CH,), jnp.int32),
            pltpu.VMEM((_RPT * _CH,), jnp.int32),
            pltpu.VMEM((_RCH * _CH, _D_HID), jnp.float32),
            pltpu.VMEM((_RCH * _CH, _D_HID), jnp.float32),
            pltpu.VMEM((_ZCH, _D_HID), jnp.float32),
            pltpu.VMEM_SHARED((max_rows, _D_HID), jnp.float32),
            pltpu.SemaphoreType.DMA,
            pltpu.SemaphoreType.DMA,
        ],
        compiler_params=pltpu.CompilerParams(use_tc_tiling_on_sc=False),
    )
    return call(*ztabs_flat, gidx, sdst, zrows)


def _deg_body(degsrc, degdst, ones, zrows8, *rest):
    outs = rest[:18]
    idx_v, ones_v, zeros_v, spmem = rest[18:]
    cid = lax.axis_index("c")
    sid = lax.axis_index("s")
    pltpu.sync_copy(ones, ones_v)
    pltpu.sync_copy(zrows8, zeros_v)

    for core in range(_NC):
        idx_hbm = degsrc if core == 0 else degdst

        @pl.when(cid == core)
        def _(core=core, idx_hbm=idx_hbm):
            for e in range(9):
                s, d = _ETYPES[e]
                n = _N_TYPES[s] if core == 0 else _N_TYPES[d]
                rows = _deg_rows(n)
                q = rows // _NS
                out = outs[core * 9 + e]
                pltpu.sync_copy(zeros_v.at[pl.ds(0, q)],
                                spmem.at[pl.ds(sid * q, q)])
                plsc.subcore_barrier()
                pltpu.sync_copy(idx_hbm.at[e, sid], idx_v)
                ec = _RCH * _CH

                def chunk(j, carry):
                    pltpu.sync_copy(
                        ones_v, spmem.at[idx_v.at[pl.ds(j * ec, ec)]],
                        add=True)
                    return carry

                lax.fori_loop(0, _RPT // _RCH, chunk, 0, unroll=False)
                plsc.subcore_barrier()
                pltpu.sync_copy(spmem.at[pl.ds(sid * q, q)],
                                out.at[pl.ds(sid * q, q)])
                plsc.subcore_barrier()


def _sc_degrees(degsrc, degdst, ones, zrows8):
    out_type = []
    for core in range(_NC):
        for e in range(9):
            s, d = _ETYPES[e]
            n = _N_TYPES[s] if core == 0 else _N_TYPES[d]
            out_type.append(jax.ShapeDtypeStruct((_deg_rows(n), 8), jnp.float32))
    mesh = plsc.VectorSubcoreMesh(
        core_axis_name="c", subcore_axis_name="s", num_cores=_NC,
        num_subcores=_NS)
    max_q = max(_deg_rows(n) for n in _N_TYPES) // _NS
    call = pl.kernel(
        _deg_body,
        out_type=tuple(out_type),
        mesh=mesh,
        scratch_types=[
            pltpu.VMEM((_RPT * _CH,), jnp.int32),
            pltpu.VMEM((_RCH * _CH, 8), jnp.float32),
            pltpu.VMEM((max_q, 8), jnp.float32),
            pltpu.VMEM_SHARED((max(_deg_rows(n) for n in _N_TYPES), 8),
                              jnp.float32),
        ],
        compiler_params=pltpu.CompilerParams(use_tc_tiling_on_sc=False),
    )
    return call(degsrc, degdst, ones, zrows8)


_B = 1000


def _b0(*block):
    return pl.BlockSpec(block, lambda i: (0,) * len(block))


def _brow(r, *block):
    return pl.BlockSpec(block, lambda i, r=r: (r,) + (0,) * (len(block) - 1))


def _a_tail(h, w_refs, ocnt_refs, out_refs):
    for k in range(3):
        t = jnp.dot(h, w_refs[k][0, 0],
                    preferred_element_type=jnp.float32)
        s = lax.rsqrt(jnp.maximum(ocnt_refs[k][...][:, :1], 1.0))
        out_refs[k][...] = t * s


def _a_specs_args(l, s, Wc, outdeg):
    in_specs, args = [], []
    for e in _SRC_GROUPS[s]:
        in_specs.append(pl.BlockSpec(
            (1, 1, _D_HID, _D_HID), lambda i, l=l, e=e: (l, e, 0, 0)))
        args.append(Wc)
    for e in _SRC_GROUPS[s]:
        in_specs.append(pl.BlockSpec((_B, 8), lambda i: (i, 0)))
        args.append(outdeg[e])
    return in_specs, args


def _embed_a_body(s, x_ref, wemb_ref, bemb_ref, w0, w1, w2, c0, c1, c2,
                  o0, o1, o2):
    h = jnp.dot(x_ref[...], wemb_ref[0],
                preferred_element_type=jnp.float32)
    h = h + bemb_ref[s:s + 1, :]
    _a_tail(h, (w0, w1, w2), (c0, c1, c2), (o0, o1, o2))


def _z_out(n):
    return (
        [pl.BlockSpec((_B, _D_HID), lambda i: (i, 0))] * 3,
        [jax.ShapeDtypeStruct((n, _D_HID), jnp.float32)] * 3,
    )


def _embed_a(x, s, W_emb, b_emb, Wc, outdeg):
    n = x.shape[0]
    in_specs = [pl.BlockSpec((_B, _D_IN), lambda i: (i, 0)),
                _brow(s, 1, _D_IN, _D_HID),
                _b0(3, _D_HID)]
    a_specs, a_args = _a_specs_args(0, s, Wc, outdeg)
    in_specs += a_specs
    out_specs, out_shape = _z_out(n)
    return pl.pallas_call(
        functools.partial(_embed_a_body, s),
        grid=(n // _B,),
        in_specs=in_specs,
        out_specs=out_specs,
        out_shape=out_shape,
    )(x, W_emb, b_emb, *a_args)


def _c_mid(ncontrib, bias_rows, d, refs):
    agg_refs = refs[:ncontrib]
    cnt_refs = refs[ncontrib:2 * ncontrib]
    k = 2 * ncontrib
    bc_ref, g_ref, b_ref = refs[k], refs[k + 1], refs[k + 2]
    acc = jnp.zeros((_B, _D_HID), jnp.float32)
    for i in range(ncontrib):
        s = lax.rsqrt(jnp.maximum(cnt_refs[i][...][:, :1], 1.0))
        acc = acc + agg_refs[i][...] * s
    bias = jnp.zeros((1, _D_HID), jnp.float32)
    for r in bias_rows:
        bias = bias + bc_ref[0, r:r + 1, :]
    acc = acc + bias
    mu = jnp.mean(acc, axis=-1, keepdims=True)
    var = jnp.mean((acc - mu) ** 2, axis=-1, keepdims=True)
    y = (acc - mu) * lax.rsqrt(var + 1e-5) * g_ref[d:d + 1, :] \
        + b_ref[d:d + 1, :]
    return jnp.maximum(y, 0.0), 2 * ncontrib + 3


def _fused_ca_body(ncontrib, bias_rows, d, *refs):
    y, k = _c_mid(ncontrib, bias_rows, d, refs)
    w_refs = refs[k:k + 3]
    ocnt_refs = refs[k + 3:k + 6]
    out_refs = refs[k + 6:k + 9]
    _a_tail(y, w_refs, ocnt_refs, out_refs)


def _final_c_body(ncontrib, bias_rows, d, *refs):
    y, k = _c_mid(ncontrib, bias_rows, d, refs)
    wcls_ref, bcls_ref, out_ref = refs[k], refs[k + 1], refs[k + 2]
    y = jnp.dot(y, wcls_ref[0], preferred_element_type=jnp.float32)
    out_ref[...] = y + bcls_ref[d:d + 1, :]


def _c_specs_args(l, aggs, cnts, bc, ln_g, ln_b):
    in_specs = []
    args = []
    for a in aggs:
        in_specs.append(pl.BlockSpec((_B, _D_HID), lambda i: (i, 0)))
        args.append(a)
    for c in cnts:
        in_specs.append(pl.BlockSpec((_B, 8), lambda i: (i, 0)))
        args.append(c)
    in_specs += [_brow(l, 1, 9, _D_HID), _b0(3, _D_HID), _b0(3, _D_HID)]
    args += [bc, ln_g, ln_b]
    return in_specs, args


def _fused_ca(n, l, d, aggs, cnts, bc, bias_rows, ln_g, ln_b, Wc, outdeg):
    in_specs, args = _c_specs_args(l, aggs, cnts, bc, ln_g, ln_b)
    a_specs, a_args = _a_specs_args(l + 1, d, Wc, outdeg)
    in_specs += a_specs
    args += a_args
    out_specs, out_shape = _z_out(n)
    return pl.pallas_call(
        functools.partial(_fused_ca_body, len(aggs), tuple(bias_rows), d),
        grid=(n // _B,),
        in_specs=in_specs,
        out_specs=out_specs,
        out_shape=out_shape,
    )(*args)


def _final_c(n, l, d, aggs, cnts, bc, bias_rows, ln_g, ln_b, Wcls, bcls):
    in_specs, args = _c_specs_args(l, aggs, cnts, bc, ln_g, ln_b)
    in_specs += [_brow(d, 1, _D_HID, _D_OUT), _b0(3, _D_OUT)]
    args += [Wcls, bcls]
    return pl.pallas_call(
        functools.partial(_final_c_body, len(aggs), tuple(bias_rows), d),
        grid=(n // _B,),
        in_specs=in_specs,
        out_specs=pl.BlockSpec((_B, _D_OUT), lambda i: (i, 0)),
        out_shape=jax.ShapeDtypeStruct((n, _D_OUT), jnp.float32),
    )(*args)


def kernel(assmpt_feat, rule_feat, non_assmpt_feat, W_emb, b_emb, Wc, bc,
           ln_g, ln_b, Wcls, bcls, edges_src, edges_dst):
    feats = (assmpt_feat, rule_feat, non_assmpt_feat)
    npad = _EPAD - _E

    gidx_l, degsrc_l, sdst_l = [], [], []
    for e, (s, d) in enumerate(_ETYPES):
        n_s, n_d = _N_TYPES[s], _N_TYPES[d]
        src_e, dst_e = edges_src[e], edges_dst[e]
        gidx_l.append(jnp.concatenate(
            [src_e, jnp.zeros((npad,), jnp.int32)]))
        degsrc_l.append(jnp.concatenate(
            [src_e, jnp.full((npad,), n_s, jnp.int32)]))
        sdst_l.append(jnp.concatenate(
            [dst_e, jnp.full((npad,), n_d, jnp.int32)]))
    gidx = jnp.stack(gidx_l).reshape(9, _NS, _RPT * _CH)
    degsrc = jnp.stack(degsrc_l).reshape(9, _NS, _RPT * _CH)
    sdst = jnp.stack(sdst_l).reshape(9, _NS, _RPT * _CH)
    ones8 = jnp.ones((_RCH * _CH, 8), jnp.float32)
    max_q8 = max(_deg_rows(n) for n in _N_TYPES) // _NS
    zrows8 = jnp.zeros((max_q8, 8), jnp.float32)
    zrows = jnp.zeros((_ZCH, _D_HID), jnp.float32)

    degs = _sc_degrees(degsrc, sdst, ones8, zrows8)
    outdeg = degs[:9]
    indeg = degs[9:]

    ztabs = [_embed_a(feats[s], s, W_emb, b_emb, Wc, outdeg)
             for s in range(3)]

    out = [None, None, None]
    for l in range(_N_LAYERS):
        aggs = _sc_feat_scatter(
            [t for zs in ztabs for t in zs], gidx, sdst, zrows)
        last = l == _N_LAYERS - 1
        nxt = []
        for d in range(3):
            n_d = _N_TYPES[d]
            contribs = _DST_CONTRIBS[d]
            a_full = [aggs[j] for j, _ in contribs]
            c_full = [indeg[e] for _, e in contribs]
            if last:
                out[d] = _final_c(
                    n_d, l, d, a_full, c_full, bc, _DST_GROUPS[d],
                    ln_g, ln_b, Wcls, bcls)
            else:
                nxt.append(_fused_ca(
                    n_d, l, d, a_full, c_full, bc, _DST_GROUPS[d],
                    ln_g, ln_b, Wc, outdeg))
        ztabs = nxt

    return out[0], out[1], out[2]
